# Initial kernel scaffold; baseline (speedup 1.0000x reference)
import jax
import jax.numpy as jnp
from jax.experimental import pallas as pl


def kernel(x, bfs_indices, bfs_parents, in_proj_w, conv_w, conv_b, x_proj_weight, dt_projs_weight, dt_projs_bias, A_logs, Ds, out_norm_g, out_norm_b, h_norm_g, h_norm_b, out_proj_w):
    raise NotImplementedError("write your pallas kernel here")


# speedup vs baseline: 141.9955x; 141.9955x over previous
import functools

import jax
import jax.numpy as jnp
from jax import lax
from jax.experimental import pallas as pl
from jax.experimental.pallas import tpu as pltpu
from jax.experimental.pallas import tpu_sc as plsc

L = 196
LP = 208
D = 768
DC = 192
NK = 4
HW = 14


def _silu(v):
    return v * jax.nn.sigmoid(v)


def _softplus(v):
    return jnp.maximum(v, 0.0) + jnp.log1p(jnp.exp(-jnp.abs(v)))


def _ln(v, g, b, eps=1e-5):
    m = jnp.mean(v, axis=-1, keepdims=True)
    var = jnp.mean((v - m) * (v - m), axis=-1, keepdims=True)
    return (v - m) * lax.rsqrt(var + eps) * g + b


def _pre_body(x_ref, ipw_ref, w9_ref, cb_ref, xpw_ref, dtw_ref, dtb_ref,
              alog_ref, dbx_ref, da_ref, xc_ref, cc_ref, z_ref):
    xb = x_ref[0]
    xz = jnp.dot(xb, ipw_ref[...], preferred_element_type=jnp.float32)
    xp = xz[:, :D]
    z_ref[0] = xz[:, D:]

    zpad = jnp.zeros((15, D), jnp.float32)
    xpad = jnp.concatenate([zpad, xp, zpad], axis=0)
    lidx = lax.broadcasted_iota(jnp.int32, (L, 1), 0)
    wmod = lidx % HW
    hdiv = lidx // HW
    acc = jnp.zeros((L, D), jnp.float32)
    for di in (-1, 0, 1):
        for dj in (-1, 0, 1):
            s = di * HW + dj
            shifted = xpad[15 + s:15 + s + L, :]
            mask = None
            if dj == 1:
                mask = wmod != (HW - 1)
            elif dj == -1:
                mask = wmod != 0
            if di == 1:
                mh = hdiv != (HW - 1)
                mask = mh if mask is None else (mask & mh)
            elif di == -1:
                mh = hdiv != 0
                mask = mh if mask is None else (mask & mh)
            if mask is not None:
                shifted = jnp.where(mask, shifted, 0.0)
            kidx = (di + 1) * 3 + (dj + 1)
            acc = acc + shifted * w9_ref[kidx:kidx + 1, :]
    xc = _silu(acc + cb_ref[...])
    xc_ref[0] = xc

    xdbl = jnp.dot(xc, xpw_ref[...], preferred_element_type=jnp.float32)
    bs = xdbl[:, 24:25]
    cc_ref[0] = xdbl[:, 25:26]
    dts = jnp.dot(xdbl[:, :24], dtw_ref[...], preferred_element_type=jnp.float32)
    dts = _softplus(dts + dtb_ref[...])
    da = jnp.exp(dts * jnp.exp(alog_ref[...]))
    dbx = dts * bs * xc
    for k in range(NK):
        dbx_ref[0, k] = dbx[:, k * DC:(k + 1) * DC]
        da_ref[0, k] = da[:, k * DC:(k + 1) * DC]


def _sc_tree(dbx_hbm, da_hbm, si_hbm, sp_hbm, out_hbm, fx_v, ea_v, hb_v,
             si_v, sp_v):
    c = lax.axis_index("c")
    s = lax.axis_index("s")
    wid = s * 2 + c
    b = wid % 8
    k = wid // 8
    pltpu.sync_copy(dbx_hbm.at[b, k], fx_v)
    pltpu.sync_copy(da_hbm.at[b, k], ea_v)
    pltpu.sync_copy(si_hbm.at[b], si_v)
    pltpu.sync_copy(sp_hbm.at[b], sp_v)

    lanes = lax.iota(jnp.int32, 16)
    zero16 = jnp.zeros((16,), jnp.float32)

    def step(t, carry):
        tv = jnp.full((16,), t, jnp.int32)
        nid = plsc.load_gather(si_v, [tv])
        par = plsc.load_gather(sp_v, [tv])
        valid = (par >= 0) & (par < tv)
        spar = jnp.maximum(par, 0)
        for cc in range(DC // 16):
            ln = lanes + (cc * 16)
            fx = plsc.load_gather(fx_v, [nid, ln])
            ea = plsc.load_gather(ea_v, [nid, ln])
            hp = plsc.load_gather(hb_v, [spar, ln])
            hp = jnp.where(valid, hp, 0.0)
            hb_v[t, pl.ds(cc * 16, 16)] = ea * hp + fx
        return carry

    lax.fori_loop(0, L, step, 0, unroll=False)

    def zrow(r, carry):
        for cc in range(DC // 16):
            fx_v[r, pl.ds(cc * 16, 16)] = zero16
        return carry

    lax.fori_loop(0, L, zrow, 0, unroll=False)

    def scat(t, carry):
        tv = jnp.full((16,), t, jnp.int32)
        nid = plsc.load_gather(si_v, [tv])
        for cc in range(DC // 16):
            row = hb_v[t, pl.ds(cc * 16, 16)]
            plsc.store_scatter(fx_v, [nid, lanes + cc * 16], row)
        return carry

    lax.fori_loop(0, L, scat, 0, unroll=False)
    pltpu.sync_copy(fx_v, out_hbm.at[b, k])


def _post_body(h4_ref, cc_ref, xc_ref, z_ref, ds_ref, ong_ref, onb_ref,
               hng_ref, hnb_ref, opw_ref, out_ref):
    h = jnp.concatenate([h4_ref[0, k] for k in range(NK)], axis=1)
    hn = _ln(h, hng_ref[...], hnb_ref[...])
    y = hn * cc_ref[0] + ds_ref[...] * xc_ref[0]
    y = _ln(y, ong_ref[...], onb_ref[...])
    y = y * _silu(z_ref[0])
    out_ref[0] = jnp.dot(y, opw_ref[...], preferred_element_type=jnp.float32)


@jax.jit
def kernel(x, bfs_indices, bfs_parents, in_proj_w, conv_w, conv_b,
           x_proj_weight, dt_projs_weight, dt_projs_bias, A_logs, Ds,
           out_norm_g, out_norm_b, h_norm_g, h_norm_b, out_proj_w):
    Bn, Hn, Wn, dm = x.shape
    xf = x.reshape(Bn, L, dm)
    ipwT = in_proj_w.T
    w9 = conv_w.reshape(D, 9).T
    cb = conv_b.reshape(1, D)
    xpwT = x_proj_weight[0].T
    dtwT = dt_projs_weight[0].T
    dtb = dt_projs_bias.reshape(1, D)
    alog = A_logs.reshape(1, D)
    ds2 = Ds.reshape(1, D)
    ong = out_norm_g.reshape(1, D)
    onb = out_norm_b.reshape(1, D)
    hng = h_norm_g.reshape(1, D)
    hnb = h_norm_b.reshape(1, D)
    opwT = out_proj_w.T
    si_pad = jnp.pad(bfs_indices.astype(jnp.int32), ((0, 0), (0, LP - L)))
    sp_pad = jnp.pad(bfs_parents.astype(jnp.int32), ((0, 0), (0, LP - L)))

    f32 = jnp.float32
    rep2 = lambda b: (0, 0)
    dbx4, da4, xcv, ccv, zv = pl.pallas_call(
        _pre_body,
        grid=(Bn,),
        in_specs=[
            pl.BlockSpec((1, L, dm), lambda b: (b, 0, 0)),
            pl.BlockSpec((dm, 2 * D), rep2),
            pl.BlockSpec((9, D), rep2),
            pl.BlockSpec((1, D), rep2),
            pl.BlockSpec((D, 26), rep2),
            pl.BlockSpec((24, D), rep2),
            pl.BlockSpec((1, D), rep2),
            pl.BlockSpec((1, D), rep2),
        ],
        out_specs=[
            pl.BlockSpec((1, NK, L, DC), lambda b: (b, 0, 0, 0)),
            pl.BlockSpec((1, NK, L, DC), lambda b: (b, 0, 0, 0)),
            pl.BlockSpec((1, L, D), lambda b: (b, 0, 0)),
            pl.BlockSpec((1, L, 1), lambda b: (b, 0, 0)),
            pl.BlockSpec((1, L, D), lambda b: (b, 0, 0)),
        ],
        out_shape=[
            jax.ShapeDtypeStruct((Bn, NK, L, DC), f32),
            jax.ShapeDtypeStruct((Bn, NK, L, DC), f32),
            jax.ShapeDtypeStruct((Bn, L, D), f32),
            jax.ShapeDtypeStruct((Bn, L, 1), f32),
            jax.ShapeDtypeStruct((Bn, L, D), f32),
        ],
    )(xf, ipwT, w9, cb, xpwT, dtwT, dtb, alog)

    sc_call = pl.kernel(
        _sc_tree,
        out_type=jax.ShapeDtypeStruct((Bn, NK, L, DC), f32),
        mesh=plsc.VectorSubcoreMesh(core_axis_name="c", subcore_axis_name="s",
                                    num_cores=2, num_subcores=16),
        compiler_params=pltpu.CompilerParams(needs_layout_passes=False,
                                             use_tc_tiling_on_sc=False),
        scratch_types=[
            pltpu.VMEM((L, DC), f32),
            pltpu.VMEM((L, DC), f32),
            pltpu.VMEM((L, DC), f32),
            pltpu.VMEM((LP,), jnp.int32),
            pltpu.VMEM((LP,), jnp.int32),
        ],
    )
    h4 = sc_call(dbx4, da4, si_pad, sp_pad)

    y = pl.pallas_call(
        _post_body,
        grid=(Bn,),
        in_specs=[
            pl.BlockSpec((1, NK, L, DC), lambda b: (b, 0, 0, 0)),
            pl.BlockSpec((1, L, 1), lambda b: (b, 0, 0)),
            pl.BlockSpec((1, L, D), lambda b: (b, 0, 0)),
            pl.BlockSpec((1, L, D), lambda b: (b, 0, 0)),
            pl.BlockSpec((1, D), rep2),
            pl.BlockSpec((1, D), rep2),
            pl.BlockSpec((1, D), rep2),
            pl.BlockSpec((1, D), rep2),
            pl.BlockSpec((1, D), rep2),
            pl.BlockSpec((D, dm), rep2),
        ],
        out_specs=pl.BlockSpec((1, L, dm), lambda b: (b, 0, 0)),
        out_shape=jax.ShapeDtypeStruct((Bn, L, dm), f32),
    )(h4, ccv, xcv, zv, ds2, ong, onb, hng, hnb, opwT)

    return y.reshape(Bn, Hn, Wn, dm)
